# baseline (speedup 1.0000x reference)
import dataclasses

import jax
import jax.numpy as jnp
from jax import lax
from jax.experimental import pallas as pl
from jax.experimental.pallas import tpu as pltpu
from jax.experimental.pallas import tpu_sc as plsc

N = 10000
E = 80000
D = 128
R = 4

NC = 2
NS = 16
NW = NC * NS
CH = 128
NCH = 80
SUP = 16
NSUP = NCH // SUP
PER_W = NCH * CH
NP = 10240
RPS = NP // NS

BN = 2000


def _xl_body(x_ref, w_ref, o_ref):
    o_ref[...] = lax.dot_general(
        x_ref[...], w_ref[...], (((1,), (1,)), ((), ())),
        preferred_element_type=jnp.float32)


def _finish_body(p_ref, o_ref):
    o_ref[...] = jnp.tanh(p_ref[0] + p_ref[1])


def _sc_agg(meta_hbm, xl_hbm, out_hbm,
            meta_v, rows0, rows1, acc_sh, gsem0, gsem1):
    c = lax.axis_index("c")
    s = lax.axis_index("s")
    wid = s * NC + c

    @pl.loop(0, CH)
    def _(r):
        for j in range(D // 16):
            rows0[r, pl.ds(j * 16, 16)] = jnp.zeros((16,), jnp.float32)

    zbase = s * RPS
    for t in range(RPS // CH):
        pltpu.sync_copy(rows0, acc_sh.at[pl.ds(zbase + t * CH, CH)])
    plsc.subcore_barrier()

    H = CH // 2

    def fetch(rows_v, sem, ch):
        pltpu.async_copy(
            xl_hbm.at[meta_v.at[ch, pl.ds(0, H)]],
            rows_v.at[pl.ds(0, H)], sem)
        pltpu.async_copy(
            xl_hbm.at[meta_v.at[ch, pl.ds(H, H)]],
            rows_v.at[pl.ds(H, H)], sem)

    def process(rows_v, sem, ch):
        pltpu.make_async_copy(xl_hbm.at[pl.ds(0, CH)], rows_v, sem).wait()
        evrow = jnp.broadcast_to(2 * SUP + ch, (16,)).astype(jnp.int32)

        @plsc.parallel_loop(0, CH, unroll=4)
        def _(k):
            spl = plsc.bitcast(
                plsc.load_gather(
                    meta_v,
                    [evrow, jnp.broadcast_to(k, (16,)).astype(jnp.int32)]),
                jnp.float32)
            for j in range(D // 16):
                sl = pl.ds(j * 16, 16)
                rows_v[k, sl] = rows_v[k, sl] * spl

        pltpu.sync_copy(rows_v, acc_sh.at[meta_v.at[SUP + ch]], add=True)

    @pl.loop(0, NSUP)
    def _(sp):
        pltpu.sync_copy(meta_hbm.at[wid * NSUP + sp], meta_v)
        fetch(rows0, gsem0, 0)

        @pl.loop(0, SUP // 2 - 1)
        def _(i):
            ch = i * 2
            fetch(rows1, gsem1, ch + 1)
            process(rows0, gsem0, ch)
            fetch(rows0, gsem0, ch + 2)
            process(rows1, gsem1, ch + 1)

        fetch(rows1, gsem1, SUP - 1)
        process(rows0, gsem0, SUP - 2)
        process(rows1, gsem1, SUP - 1)

    plsc.subcore_barrier()
    pltpu.sync_copy(acc_sh.at[pl.ds(s * RPS, RPS)],
                    out_hbm.at[c, pl.ds(s * RPS, RPS)])


def kernel(x, W0, W1, W2, W3,
           edge_vals_0, edge_vals_1, edge_vals_2, edge_vals_3,
           edge_index_0, edge_index_1, edge_index_2, edge_index_3):
    eis = [edge_index_0, edge_index_1, edge_index_2, edge_index_3]
    evs = [edge_vals_0, edge_vals_1, edge_vals_2, edge_vals_3]

    wcat = jnp.concatenate([W0, W1, W2, W3], axis=0)
    xl = pl.pallas_call(
        _xl_body,
        grid=(N // BN,),
        in_specs=[pl.BlockSpec((BN, D), lambda i: (i, 0)),
                  pl.BlockSpec((R * D, D), lambda i: (0, 0))],
        out_specs=pl.BlockSpec((BN, R * D), lambda i: (i, 0)),
        out_shape=jax.ShapeDtypeStruct((N, R * D), jnp.float32),
    )(x, wcat)
    xl_flat = xl.reshape(R * N, D)

    pad = NW * PER_W - R * E
    src = jnp.concatenate(
        [eis[r][1] * R + r for r in range(R)]
        + [jnp.zeros((pad,), jnp.int32)])
    dst = jnp.concatenate(
        [eis[r][0] for r in range(R)] + [jnp.zeros((pad,), jnp.int32)])
    ev = jnp.concatenate(evs + [jnp.zeros((pad,), jnp.float32)])
    meta = jnp.stack([src, dst, lax.bitcast_convert_type(ev, jnp.int32)])
    meta = meta.reshape(3, NW, NSUP, SUP, CH).transpose(1, 2, 0, 3, 4)
    meta = meta.reshape(NW * NSUP, 3 * SUP, CH)

    mesh = plsc.VectorSubcoreMesh(core_axis_name="c", subcore_axis_name="s")
    cp = pltpu.CompilerParams()
    if "needs_layout_passes" in pltpu.CompilerParams.__dataclass_fields__:
        cp = dataclasses.replace(cp, needs_layout_passes=False)
    partials = pl.kernel(
        _sc_agg,
        mesh=mesh,
        compiler_params=cp,
        out_type=jax.ShapeDtypeStruct((NC, NP, D), jnp.float32),
        scratch_types=[
            pltpu.VMEM((3 * SUP, CH), jnp.int32),
            pltpu.VMEM((CH, D), jnp.float32),
            pltpu.VMEM((CH, D), jnp.float32),
            pltpu.VMEM_SHARED((NP, D), jnp.float32),
            pltpu.SemaphoreType.DMA,
            pltpu.SemaphoreType.DMA,
        ],
    )(meta, xl_flat)

    out = pl.pallas_call(
        _finish_body,
        grid=(N // BN,),
        in_specs=[pl.BlockSpec((NC, BN, D), lambda i: (0, i, 0))],
        out_specs=pl.BlockSpec((BN, D), lambda i: (i, 0)),
        out_shape=jax.ShapeDtypeStruct((N, D), jnp.float32),
    )(partials)
    return out
